# Initial kernel scaffold; baseline (speedup 1.0000x reference)
import jax
import jax.numpy as jnp
from jax.experimental import pallas as pl


def kernel(x, adj, W0, b0, W1, b1, W2, b2, Wh, bh, Wo, bo):
    raise NotImplementedError("write your pallas kernel here")


# speedup vs baseline: 1.5389x; 1.5389x over previous
import jax
import jax.numpy as jnp
from jax.experimental import pallas as pl
from jax.experimental.pallas import tpu as pltpu


def _sage_body(x_ref, adj_ref, wa_ref, wb_ref, bs_ref, g_ref):
    adj = adj_ref[0]
    n = adj.shape[0]
    f = wa_ref.shape[2]
    ones = jnp.ones((n, f), jnp.float32)
    deg = jnp.dot(adj, ones, preferred_element_type=jnp.float32) + 1.0
    inv = 1.0 / deg
    h = x_ref[0]
    for i in range(3):
        neigh = jnp.dot(adj, h, preferred_element_type=jnp.float32) * inv
        z = (
            jnp.dot(h, wa_ref[i], preferred_element_type=jnp.float32)
            + jnp.dot(neigh, wb_ref[i], preferred_element_type=jnp.float32)
            + bs_ref[i]
        )
        h = jnp.maximum(z, 0.0)
    g_ref[0] = jnp.max(h, axis=0, keepdims=True)


def _head_body(g_ref, wh_ref, bh_ref, wo_ref, bo_ref, o_ref):
    t = jnp.dot(g_ref[...], wh_ref[...], preferred_element_type=jnp.float32)
    t = t + bh_ref[...]
    o = jnp.dot(t, wo_ref[...], preferred_element_type=jnp.float32)
    o_ref[...] = o + bo_ref[...]


def kernel(x, adj, W0, b0, W1, b1, W2, b2, Wh, bh, Wo, bo):
    B, N, D = x.shape
    F = W0.shape[1]
    Wa = jnp.stack([W0[:D], W1[:F], W2[:F]])
    Wb = jnp.stack([W0[D:], W1[F:], W2[F:]])
    bs = jnp.stack([b0, b1, b2]).reshape(3, 1, F)

    g = pl.pallas_call(
        _sage_body,
        grid=(B,),
        in_specs=[
            pl.BlockSpec((1, N, D), lambda b: (b, 0, 0)),
            pl.BlockSpec((1, N, N), lambda b: (b, 0, 0)),
            pl.BlockSpec((3, D, F), lambda b: (0, 0, 0)),
            pl.BlockSpec((3, F, F), lambda b: (0, 0, 0)),
            pl.BlockSpec((3, 1, F), lambda b: (0, 0, 0)),
        ],
        out_specs=pl.BlockSpec((1, 1, F), lambda b: (b, 0, 0)),
        out_shape=jax.ShapeDtypeStruct((B, 1, F), jnp.float32),
        compiler_params=pltpu.CompilerParams(
            dimension_semantics=("parallel",),
            vmem_limit_bytes=60 * 1024 * 1024,
        ),
    )(x, adj, Wa, Wb, bs)

    H = Wh.shape[1]
    O = Wo.shape[1]
    out = pl.pallas_call(
        _head_body,
        in_specs=[
            pl.BlockSpec((B, F), lambda: (0, 0)),
            pl.BlockSpec((F, H), lambda: (0, 0)),
            pl.BlockSpec((1, H), lambda: (0, 0)),
            pl.BlockSpec((H, O), lambda: (0, 0)),
            pl.BlockSpec((1, O), lambda: (0, 0)),
        ],
        out_specs=pl.BlockSpec((B, O), lambda: (0, 0)),
        out_shape=jax.ShapeDtypeStruct((B, O), jnp.float32),
    )(g.reshape(B, F), Wh, bh.reshape(1, -1), Wo, bo.reshape(1, -1))
    return out


# speedup vs baseline: 1.7868x; 1.1611x over previous
import jax
import jax.numpy as jnp
from jax.experimental import pallas as pl
from jax.experimental.pallas import tpu as pltpu


def _sage_body(x_ref, adj_ref, w_ref, bs_ref, g_ref, cat_ref):
    adj = adj_ref[0]
    n = adj.shape[0]
    f = w_ref.shape[2]
    rhs0 = jnp.concatenate([x_ref[0], jnp.ones((n, f), jnp.float32)], axis=1)
    r0 = jnp.dot(adj, rhs0, preferred_element_type=jnp.float32)
    adjb = adj.astype(jnp.bfloat16)
    inv = 1.0 / (r0[:, f:] + 1.0)
    cat_ref[:, :f] = x_ref[0].astype(jnp.bfloat16)
    cat_ref[:, f:] = (r0[:, :f] * inv).astype(jnp.bfloat16)
    h = jnp.maximum(
        jnp.dot(cat_ref[...], w_ref[0], preferred_element_type=jnp.float32)
        + bs_ref[0],
        0.0,
    )
    for i in (1, 2):
        hb = h.astype(jnp.bfloat16)
        neigh = jnp.dot(adjb, hb, preferred_element_type=jnp.float32) * inv
        cat_ref[:, :f] = hb
        cat_ref[:, f:] = neigh.astype(jnp.bfloat16)
        h = jnp.maximum(
            jnp.dot(cat_ref[...], w_ref[i], preferred_element_type=jnp.float32)
            + bs_ref[i],
            0.0,
        )
    g_ref[0] = jnp.max(h, axis=0, keepdims=True)


def _head_body(g_ref, wh_ref, bh_ref, wo_ref, bo_ref, o_ref):
    t = jnp.dot(g_ref[...], wh_ref[...], preferred_element_type=jnp.float32)
    t = t + bh_ref[...]
    o = jnp.dot(t, wo_ref[...], preferred_element_type=jnp.float32)
    o_ref[...] = o + bo_ref[...]


def kernel(x, adj, W0, b0, W1, b1, W2, b2, Wh, bh, Wo, bo):
    B, N, D = x.shape
    F = W0.shape[1]
    W = jnp.stack([W0, W1, W2]).astype(jnp.bfloat16)
    bs = jnp.stack([b0, b1, b2]).reshape(3, 1, F)

    g = pl.pallas_call(
        _sage_body,
        grid=(B,),
        in_specs=[
            pl.BlockSpec((1, N, D), lambda b: (b, 0, 0)),
            pl.BlockSpec((1, N, N), lambda b: (b, 0, 0)),
            pl.BlockSpec((3, 2 * F, F), lambda b: (0, 0, 0)),
            pl.BlockSpec((3, 1, F), lambda b: (0, 0, 0)),
        ],
        out_specs=pl.BlockSpec((1, 1, F), lambda b: (b, 0, 0)),
        out_shape=jax.ShapeDtypeStruct((B, 1, F), jnp.float32),
        scratch_shapes=[pltpu.VMEM((N, 2 * F), jnp.bfloat16)],
        compiler_params=pltpu.CompilerParams(
            dimension_semantics=("parallel",),
            vmem_limit_bytes=60 * 1024 * 1024,
        ),
    )(x, adj, W, bs)

    H = Wh.shape[1]
    O = Wo.shape[1]
    out = pl.pallas_call(
        _head_body,
        in_specs=[
            pl.BlockSpec((B, F), lambda: (0, 0)),
            pl.BlockSpec((F, H), lambda: (0, 0)),
            pl.BlockSpec((1, H), lambda: (0, 0)),
            pl.BlockSpec((H, O), lambda: (0, 0)),
            pl.BlockSpec((1, O), lambda: (0, 0)),
        ],
        out_specs=pl.BlockSpec((B, O), lambda: (0, 0)),
        out_shape=jax.ShapeDtypeStruct((B, O), jnp.float32),
    )(g.reshape(B, F), Wh, bh.reshape(1, -1), Wo, bo.reshape(1, -1))
    return out


# speedup vs baseline: 1.8446x; 1.0324x over previous
import jax
import jax.numpy as jnp
from jax.experimental import pallas as pl
from jax.experimental.pallas import tpu as pltpu

_CHUNKS = 16


def _sage_body(
    x_ref, adj_ref, wa_ref, wb_ref, bs_ref, g_ref,
    adjb_ref, h0_ref, n0_ref, h1_ref, n1_ref,
):
    adj = adj_ref[0]
    n = adj.shape[0]
    f = wa_ref.shape[2]
    c = n // _CHUNKS
    rhs0 = jnp.concatenate([x_ref[0], jnp.ones((n, f), jnp.float32)], axis=1)
    r0 = jnp.dot(adj, rhs0, preferred_element_type=jnp.float32)
    adjb_ref[...] = adj.astype(jnp.bfloat16)
    inv = 1.0 / (r0[:, f:] + 1.0)
    h0_ref[...] = x_ref[0].astype(jnp.bfloat16)
    n0_ref[...] = (r0[:, :f] * inv).astype(jnp.bfloat16)

    hA, nA, hB, nB = h0_ref, n0_ref, h1_ref, n1_ref
    for i in (0, 1):
        for k in range(_CHUNKS):
            rows = pl.ds(k * c, c)
            z = (
                jnp.dot(hA[rows, :], wa_ref[i], preferred_element_type=jnp.float32)
                + jnp.dot(nA[rows, :], wb_ref[i], preferred_element_type=jnp.float32)
                + bs_ref[i]
            )
            hB[rows, :] = jnp.maximum(z, 0.0).astype(jnp.bfloat16)
        hb = hB[...]
        for k in range(_CHUNKS):
            rows = pl.ds(k * c, c)
            nr = jnp.dot(adjb_ref[rows, :], hb, preferred_element_type=jnp.float32)
            nB[rows, :] = (nr * inv[k * c : (k + 1) * c, :]).astype(jnp.bfloat16)
        hA, nA, hB, nB = hB, nB, hA, nA
    m = None
    for k in range(_CHUNKS):
        rows = pl.ds(k * c, c)
        z = (
            jnp.dot(hA[rows, :], wa_ref[2], preferred_element_type=jnp.float32)
            + jnp.dot(nA[rows, :], wb_ref[2], preferred_element_type=jnp.float32)
            + bs_ref[2]
        )
        h = jnp.maximum(z, 0.0)
        hm = jnp.max(h, axis=0, keepdims=True)
        m = hm if m is None else jnp.maximum(m, hm)
    g_ref[0] = m


def _head_body(g_ref, wh_ref, bh_ref, wo_ref, bo_ref, o_ref):
    t = jnp.dot(g_ref[...], wh_ref[...], preferred_element_type=jnp.float32)
    t = t + bh_ref[...]
    o = jnp.dot(t, wo_ref[...], preferred_element_type=jnp.float32)
    o_ref[...] = o + bo_ref[...]


def kernel(x, adj, W0, b0, W1, b1, W2, b2, Wh, bh, Wo, bo):
    B, N, D = x.shape
    F = W0.shape[1]
    Wa = jnp.stack([W0[:D], W1[:F], W2[:F]]).astype(jnp.bfloat16)
    Wb = jnp.stack([W0[D:], W1[F:], W2[F:]]).astype(jnp.bfloat16)
    bs = jnp.stack([b0, b1, b2]).reshape(3, 1, F)

    g = pl.pallas_call(
        _sage_body,
        grid=(B,),
        in_specs=[
            pl.BlockSpec((1, N, D), lambda b: (b, 0, 0)),
            pl.BlockSpec((1, N, N), lambda b: (b, 0, 0)),
            pl.BlockSpec((3, F, F), lambda b: (0, 0, 0)),
            pl.BlockSpec((3, F, F), lambda b: (0, 0, 0)),
            pl.BlockSpec((3, 1, F), lambda b: (0, 0, 0)),
        ],
        out_specs=pl.BlockSpec((1, 1, F), lambda b: (b, 0, 0)),
        out_shape=jax.ShapeDtypeStruct((B, 1, F), jnp.float32),
        scratch_shapes=[
            pltpu.VMEM((N, N), jnp.bfloat16),
            pltpu.VMEM((N, F), jnp.bfloat16),
            pltpu.VMEM((N, F), jnp.bfloat16),
            pltpu.VMEM((N, F), jnp.bfloat16),
            pltpu.VMEM((N, F), jnp.bfloat16),
        ],
        compiler_params=pltpu.CompilerParams(
            dimension_semantics=("parallel",),
            vmem_limit_bytes=60 * 1024 * 1024,
        ),
    )(x, adj, Wa, Wb, bs)

    H = Wh.shape[1]
    O = Wo.shape[1]
    out = pl.pallas_call(
        _head_body,
        in_specs=[
            pl.BlockSpec((B, F), lambda: (0, 0)),
            pl.BlockSpec((F, H), lambda: (0, 0)),
            pl.BlockSpec((1, H), lambda: (0, 0)),
            pl.BlockSpec((H, O), lambda: (0, 0)),
            pl.BlockSpec((1, O), lambda: (0, 0)),
        ],
        out_specs=pl.BlockSpec((B, O), lambda: (0, 0)),
        out_shape=jax.ShapeDtypeStruct((B, O), jnp.float32),
    )(g.reshape(B, F), Wh, bh.reshape(1, -1), Wo, bo.reshape(1, -1))
    return out


# speedup vs baseline: 2.5958x; 1.4073x over previous
import jax
import jax.numpy as jnp
from jax.experimental import pallas as pl
from jax.experimental.pallas import tpu as pltpu

_CHUNKS = 16


def _sage_body(
    x_ref, adj_ref, wa_ref, wb_ref, bs_ref, g_ref,
    adjb_ref, inv_ref, h0_ref, n0_ref, h1_ref, n1_ref,
):
    n = adj_ref.shape[1]
    f = wa_ref.shape[2]
    c = n // _CHUNKS
    h0_ref[...] = x_ref[0].astype(jnp.bfloat16)
    rhs0 = jnp.concatenate(
        [h0_ref[...], jnp.ones((n, f), jnp.bfloat16)], axis=1
    )
    for k in range(_CHUNKS):
        rows = pl.ds(k * c, c)
        adjb_ref[rows, :] = adj_ref[0, k * c : (k + 1) * c, :].astype(jnp.bfloat16)
        r0 = jnp.dot(adjb_ref[rows, :], rhs0, preferred_element_type=jnp.float32)
        iv = 1.0 / (r0[:, f:] + 1.0)
        inv_ref[rows, :] = iv
        n0_ref[rows, :] = (r0[:, :f] * iv).astype(jnp.bfloat16)

    inv = inv_ref[...]
    hA, nA, hB, nB = h0_ref, n0_ref, h1_ref, n1_ref
    for i in (0, 1):
        for k in range(_CHUNKS):
            rows = pl.ds(k * c, c)
            z = (
                jnp.dot(hA[rows, :], wa_ref[i], preferred_element_type=jnp.float32)
                + jnp.dot(nA[rows, :], wb_ref[i], preferred_element_type=jnp.float32)
                + bs_ref[i]
            )
            hB[rows, :] = jnp.maximum(z, 0.0).astype(jnp.bfloat16)
        hb = hB[...]
        for k in range(_CHUNKS):
            rows = pl.ds(k * c, c)
            nr = jnp.dot(adjb_ref[rows, :], hb, preferred_element_type=jnp.float32)
            nB[rows, :] = (nr * inv[k * c : (k + 1) * c, :]).astype(jnp.bfloat16)
        hA, nA, hB, nB = hB, nB, hA, nA
    m = None
    for k in range(_CHUNKS):
        rows = pl.ds(k * c, c)
        z = (
            jnp.dot(hA[rows, :], wa_ref[2], preferred_element_type=jnp.float32)
            + jnp.dot(nA[rows, :], wb_ref[2], preferred_element_type=jnp.float32)
            + bs_ref[2]
        )
        h = jnp.maximum(z, 0.0)
        hm = jnp.max(h, axis=0, keepdims=True)
        m = hm if m is None else jnp.maximum(m, hm)
    g_ref[0] = m


def _head_body(g_ref, wh_ref, bh_ref, wo_ref, bo_ref, o_ref):
    t = jnp.dot(g_ref[...], wh_ref[...], preferred_element_type=jnp.float32)
    t = t + bh_ref[...]
    o = jnp.dot(t, wo_ref[...], preferred_element_type=jnp.float32)
    o_ref[...] = o + bo_ref[...]


def kernel(x, adj, W0, b0, W1, b1, W2, b2, Wh, bh, Wo, bo):
    B, N, D = x.shape
    F = W0.shape[1]
    Wa = jnp.stack([W0[:D], W1[:F], W2[:F]]).astype(jnp.bfloat16)
    Wb = jnp.stack([W0[D:], W1[F:], W2[F:]]).astype(jnp.bfloat16)
    bs = jnp.stack([b0, b1, b2]).reshape(3, 1, F)

    g = pl.pallas_call(
        _sage_body,
        grid=(B,),
        in_specs=[
            pl.BlockSpec((1, N, D), lambda b: (b, 0, 0)),
            pl.BlockSpec((1, N, N), lambda b: (b, 0, 0)),
            pl.BlockSpec((3, F, F), lambda b: (0, 0, 0)),
            pl.BlockSpec((3, F, F), lambda b: (0, 0, 0)),
            pl.BlockSpec((3, 1, F), lambda b: (0, 0, 0)),
        ],
        out_specs=pl.BlockSpec((1, 1, F), lambda b: (b, 0, 0)),
        out_shape=jax.ShapeDtypeStruct((B, 1, F), jnp.float32),
        scratch_shapes=[
            pltpu.VMEM((N, N), jnp.bfloat16),
            pltpu.VMEM((N, F), jnp.float32),
            pltpu.VMEM((N, F), jnp.bfloat16),
            pltpu.VMEM((N, F), jnp.bfloat16),
            pltpu.VMEM((N, F), jnp.bfloat16),
            pltpu.VMEM((N, F), jnp.bfloat16),
        ],
        compiler_params=pltpu.CompilerParams(
            dimension_semantics=("parallel",),
            vmem_limit_bytes=60 * 1024 * 1024,
        ),
    )(x, adj, Wa, Wb, bs)

    H = Wh.shape[1]
    O = Wo.shape[1]
    out = pl.pallas_call(
        _head_body,
        in_specs=[
            pl.BlockSpec((B, F), lambda: (0, 0)),
            pl.BlockSpec((F, H), lambda: (0, 0)),
            pl.BlockSpec((1, H), lambda: (0, 0)),
            pl.BlockSpec((H, O), lambda: (0, 0)),
            pl.BlockSpec((1, O), lambda: (0, 0)),
        ],
        out_specs=pl.BlockSpec((B, O), lambda: (0, 0)),
        out_shape=jax.ShapeDtypeStruct((B, O), jnp.float32),
    )(g.reshape(B, F), Wh, bh.reshape(1, -1), Wo, bo.reshape(1, -1))
    return out


# speedup vs baseline: 2.7456x; 1.0577x over previous
import jax
import jax.numpy as jnp
from jax.experimental import pallas as pl
from jax.experimental.pallas import tpu as pltpu

_CHUNKS = 4


def _sage_body(
    x_ref, adj_ref, wa_ref, wb_ref, bs_ref, g_ref,
    adjb_ref, inv_ref, h0_ref, n0_ref, h1_ref, n1_ref,
):
    n = adj_ref.shape[1]
    f = wa_ref.shape[2]
    c = n // _CHUNKS
    h0_ref[...] = x_ref[0].astype(jnp.bfloat16)
    rhs0 = jnp.concatenate(
        [h0_ref[...], jnp.ones((n, f), jnp.bfloat16)], axis=1
    )
    for k in range(_CHUNKS):
        rows = pl.ds(k * c, c)
        adjb_ref[rows, :] = adj_ref[0, k * c : (k + 1) * c, :].astype(jnp.bfloat16)
        r0 = jnp.dot(adjb_ref[rows, :], rhs0, preferred_element_type=jnp.float32)
        iv = 1.0 / (r0[:, f:] + 1.0)
        inv_ref[rows, :] = iv
        n0_ref[rows, :] = (r0[:, :f] * iv).astype(jnp.bfloat16)

    inv = inv_ref[...]
    hA, nA, hB, nB = h0_ref, n0_ref, h1_ref, n1_ref
    for i in (0, 1):
        for k in range(_CHUNKS):
            rows = pl.ds(k * c, c)
            z = (
                jnp.dot(hA[rows, :], wa_ref[i], preferred_element_type=jnp.float32)
                + jnp.dot(nA[rows, :], wb_ref[i], preferred_element_type=jnp.float32)
                + bs_ref[i]
            )
            hB[rows, :] = jnp.maximum(z, 0.0).astype(jnp.bfloat16)
        hb = hB[...]
        for k in range(_CHUNKS):
            rows = pl.ds(k * c, c)
            nr = jnp.dot(adjb_ref[rows, :], hb, preferred_element_type=jnp.float32)
            nB[rows, :] = (nr * inv[k * c : (k + 1) * c, :]).astype(jnp.bfloat16)
        hA, nA, hB, nB = hB, nB, hA, nA
    m = None
    for k in range(_CHUNKS):
        rows = pl.ds(k * c, c)
        z = (
            jnp.dot(hA[rows, :], wa_ref[2], preferred_element_type=jnp.float32)
            + jnp.dot(nA[rows, :], wb_ref[2], preferred_element_type=jnp.float32)
            + bs_ref[2]
        )
        h = jnp.maximum(z, 0.0)
        hm = jnp.max(h, axis=0, keepdims=True)
        m = hm if m is None else jnp.maximum(m, hm)
    g_ref[0] = m


def _head_body(g_ref, wh_ref, bh_ref, wo_ref, bo_ref, o_ref):
    t = jnp.dot(g_ref[...], wh_ref[...], preferred_element_type=jnp.float32)
    t = t + bh_ref[...]
    o = jnp.dot(t, wo_ref[...], preferred_element_type=jnp.float32)
    o_ref[...] = o + bo_ref[...]


def kernel(x, adj, W0, b0, W1, b1, W2, b2, Wh, bh, Wo, bo):
    B, N, D = x.shape
    F = W0.shape[1]
    Wa = jnp.stack([W0[:D], W1[:F], W2[:F]]).astype(jnp.bfloat16)
    Wb = jnp.stack([W0[D:], W1[F:], W2[F:]]).astype(jnp.bfloat16)
    bs = jnp.stack([b0, b1, b2]).reshape(3, 1, F)

    g = pl.pallas_call(
        _sage_body,
        grid=(B,),
        in_specs=[
            pl.BlockSpec((1, N, D), lambda b: (b, 0, 0)),
            pl.BlockSpec((1, N, N), lambda b: (b, 0, 0)),
            pl.BlockSpec((3, F, F), lambda b: (0, 0, 0)),
            pl.BlockSpec((3, F, F), lambda b: (0, 0, 0)),
            pl.BlockSpec((3, 1, F), lambda b: (0, 0, 0)),
        ],
        out_specs=pl.BlockSpec((1, 1, F), lambda b: (b, 0, 0)),
        out_shape=jax.ShapeDtypeStruct((B, 1, F), jnp.float32),
        scratch_shapes=[
            pltpu.VMEM((N, N), jnp.bfloat16),
            pltpu.VMEM((N, F), jnp.float32),
            pltpu.VMEM((N, F), jnp.bfloat16),
            pltpu.VMEM((N, F), jnp.bfloat16),
            pltpu.VMEM((N, F), jnp.bfloat16),
            pltpu.VMEM((N, F), jnp.bfloat16),
        ],
        compiler_params=pltpu.CompilerParams(
            dimension_semantics=("parallel",),
            vmem_limit_bytes=60 * 1024 * 1024,
        ),
    )(x, adj, Wa, Wb, bs)

    H = Wh.shape[1]
    O = Wo.shape[1]
    out = pl.pallas_call(
        _head_body,
        in_specs=[
            pl.BlockSpec((B, F), lambda: (0, 0)),
            pl.BlockSpec((F, H), lambda: (0, 0)),
            pl.BlockSpec((1, H), lambda: (0, 0)),
            pl.BlockSpec((H, O), lambda: (0, 0)),
            pl.BlockSpec((1, O), lambda: (0, 0)),
        ],
        out_specs=pl.BlockSpec((B, O), lambda: (0, 0)),
        out_shape=jax.ShapeDtypeStruct((B, O), jnp.float32),
    )(g.reshape(B, F), Wh, bh.reshape(1, -1), Wo, bo.reshape(1, -1))
    return out
